# baseline (speedup 1.0000x reference)
import jax
import jax.numpy as jnp
import numpy as np
from jax import lax
from jax.experimental import pallas as pl
from jax.experimental.pallas import tpu as pltpu
from jax.experimental.pallas import tpu_sc as plsc

N = 10000
E = 160000
D_IN = 128
D_H = 64
DIM = 4
KS = 3
K = KS ** DIM
KPAD = 84
S = 2 ** DIM
AGG_W = 80

NC, NS = 2, 16
NW = NC * NS
EPW = E // NW
C = 40
NCHUNK = EPW // C
ROWS = C * S
IDXW = 80

HALF = 5120
AGGR = 5248
BSTAGE = AGGR // NS
CB = 80
EPS_B = E // NS
NCHUNK_B = EPS_B // CB

_PERM = np.empty(D_H, dtype=np.int32)
for _p in range(D_H):
    _g, _q = divmod(_p, 32)
    _PERM[_p] = 32 * _g + (_q // 2 if _q % 2 == 0 else 16 + _q // 2)
_MASKHI = -65536


def _matmul_kernel(xh_ref, w_ref, out_ref):
    out_ref[...] = jnp.dot(xh_ref[...], w_ref[...],
                           preferred_element_type=jnp.float32)


@jax.jit
def _matmul(xh, w2):
    BN, BM = 2000, 2688
    return pl.pallas_call(
        _matmul_kernel,
        grid=(N // BN, (KPAD * D_H) // BM),
        in_specs=[
            pl.BlockSpec((BN, D_IN + D_H), lambda i, j: (i, 0)),
            pl.BlockSpec((D_IN + D_H, BM), lambda i, j: (0, j)),
        ],
        out_specs=pl.BlockSpec((BN, BM), lambda i, j: (i, j)),
        out_shape=jax.ShapeDtypeStruct((N, KPAD * D_H), jnp.float32),
    )(xh, w2)


def _edge_prep_kernel(attr_ref, src_ref, basis_ref, gidx_ref):
    attr = attr_ref[...][:DIM]
    src = src_ref[...][0:1]
    BEL = attr.shape[1]
    s_ids = lax.broadcasted_iota(jnp.int32, (S, BEL), 0)
    v = attr * (KS - 1)
    bot = jnp.clip(jnp.floor(v), 0.0, KS - 1.0)
    frac = v - bot
    boti = bot.astype(jnp.int32)
    topi = jnp.minimum(boti + 1, KS - 1)
    basis = jnp.ones((S, BEL), jnp.float32)
    wi = jnp.zeros((S, BEL), jnp.int32)
    for i in range(DIM):
        bit = (s_ids >> i) & 1
        f = frac[i:i + 1, :]
        basis = basis * jnp.where(bit == 1, f, 1.0 - f)
        idx = jnp.where(bit == 1, topi[i:i + 1, :], boti[i:i + 1, :])
        wi = wi + idx * (KS ** i)
    basis_ref[...] = basis
    gidx_ref[...] = src * KPAD + wi


@jax.jit
def _edge_prep(attr_t, src_t):
    BEL = 6400
    return pl.pallas_call(
        _edge_prep_kernel,
        grid=(E // BEL,),
        in_specs=[
            pl.BlockSpec((8, BEL), lambda i: (0, i)),
            pl.BlockSpec((8, BEL), lambda i: (0, i)),
        ],
        out_specs=[
            pl.BlockSpec((S, BEL), lambda i: (0, i)),
            pl.BlockSpec((S, BEL), lambda i: (0, i)),
        ],
        out_shape=[
            jax.ShapeDtypeStruct((S, E), jnp.float32),
            jax.ShapeDtypeStruct((S, E), jnp.int32),
        ],
    )(attr_t, src_t)


def _sc_msg_body(gidx_hbm, basis_hbm, table_hbm, msgs_hbm,
                 idx_v, basis_v, rows_v, msg_v,
                 semg0, semg1, seml0, seml1):
    cid = lax.axis_index("c")
    sid = lax.axis_index("s")
    wid = cid * NS + sid
    semg = (semg0, semg1)
    seml = (seml0, seml1)
    NIR = ROWS // IDXW

    deg_vec = jnp.where(lax.iota(jnp.int32, 16) == 0, 1.0, 0.0)

    def _init_msg(c, _):
        msg_v[c, pl.ds(D_H, 16)] = deg_vec
        return 0
    lax.fori_loop(0, C, _init_msg, 0)

    def _fire_loads(j, q):
        e0 = wid * EPW + j * C
        pltpu.async_copy(gidx_hbm.at[:, pl.ds(e0, C)], idx_v.at[q], seml[q])
        pltpu.async_copy(basis_hbm.at[pl.ds(e0, C)], basis_v.at[q], seml[q])

    def _drain_loads(q):
        pltpu.make_async_copy(gidx_hbm.at[:, pl.ds(0, C)],
                              idx_v.at[q], seml[q]).wait()
        pltpu.make_async_copy(basis_hbm.at[pl.ds(0, C)],
                              basis_v.at[q], seml[q]).wait()

    def _fire_gathers(q):
        for i in range(S):
            pltpu.async_copy(table_hbm.at[idx_v.at[q].at[i]],
                             rows_v.at[q].at[pl.ds(i * C, C)], semg[q])

    def _drain_gathers(q):
        for i in range(S):
            pltpu.make_async_copy(table_hbm.at[idx_v.at[q].at[i]],
                                  rows_v.at[q].at[pl.ds(i * C, C)],
                                  semg[q]).wait()

    def _compute(j, p):
        def _edge(c, _):
            bvec = basis_v[p, c, pl.ds(0, S)]
            a0 = jnp.zeros((16,), jnp.float32)
            a1 = a0
            a2 = a0
            a3 = a0
            for s in range(S):
                b = bvec[s]
                r = s * C + c
                v0 = plsc.bitcast(rows_v[p, r, pl.ds(0, 32)], jnp.int32)
                v1 = plsc.bitcast(rows_v[p, r, pl.ds(32, 32)], jnp.int32)
                a0 = a0 + b * plsc.bitcast(v0 << 16, jnp.float32)
                a1 = a1 + b * plsc.bitcast(v0 & _MASKHI, jnp.float32)
                a2 = a2 + b * plsc.bitcast(v1 << 16, jnp.float32)
                a3 = a3 + b * plsc.bitcast(v1 & _MASKHI, jnp.float32)
            msg_v[c, pl.ds(0, 16)] = a0
            msg_v[c, pl.ds(16, 16)] = a1
            msg_v[c, pl.ds(32, 16)] = a2
            msg_v[c, pl.ds(48, 16)] = a3
            return 0
        lax.fori_loop(0, C, _edge, 0)
        pltpu.sync_copy(msg_v, msgs_hbm.at[pl.ds(wid * EPW + j * C, C)])

    def _body(j, p, fire_next):
        q = 1 - p
        if fire_next:
            _fire_loads(j + 1, q)
        _drain_gathers(p)
        _compute(j, p)
        if fire_next:
            _drain_loads(q)
            _fire_gathers(q)

    _fire_loads(0, 0)
    _drain_loads(0)
    _fire_gathers(0)

    def _pair(j2, _):
        j = j2 * 2
        _body(j, 0, True)
        _body(j + 1, 1, True)
        return 0
    lax.fori_loop(0, NCHUNK // 2, _pair, 0)
    _body(NCHUNK - 1, 0, False)


@jax.jit
def _sc_msg(gidx2, basis, table):
    mesh = plsc.VectorSubcoreMesh(core_axis_name="c", subcore_axis_name="s",
                                  num_cores=NC, num_subcores=NS)
    f = pl.kernel(
        _sc_msg_body,
        out_type=jax.ShapeDtypeStruct((E, AGG_W), jnp.float32),
        mesh=mesh,
        scratch_types=[
            pltpu.VMEM((2, S, C), jnp.int32),
            pltpu.VMEM((2, C, S), jnp.float32),
            pltpu.VMEM((2, ROWS, D_H), jnp.bfloat16),
            pltpu.VMEM((C, AGG_W), jnp.float32),
            pltpu.SemaphoreType.DMA,
            pltpu.SemaphoreType.DMA,
            pltpu.SemaphoreType.DMA,
            pltpu.SemaphoreType.DMA,
        ],
        compiler_params=pltpu.CompilerParams(use_tc_tiling_on_sc=False,
                                             needs_layout_passes=False),
    )
    return f(gidx2, basis, table)


def _sc_agg_body(msgs_hbm, dst_hbm, out_hbm,
                 msg_v, dst_v, idx_v, stage_v, agg_sh, semm0, semm1):
    cid = lax.axis_index("c")
    sid = lax.axis_index("s")
    base = cid * HALF
    semm = (semm0, semm1)

    zero16 = jnp.zeros((16,), jnp.float32)

    def _zero_row(i, _):
        for j in range(AGG_W // 16):
            stage_v[i, pl.ds(j * 16, 16)] = zero16
        return 0
    lax.fori_loop(0, BSTAGE, _zero_row, 0)
    pltpu.sync_copy(stage_v, agg_sh.at[pl.ds(sid * BSTAGE, BSTAGE)])
    pltpu.sync_copy(dst_hbm.at[pl.ds(sid * EPS_B, EPS_B)], dst_v)
    plsc.subcore_barrier()

    def _fire(j, q):
        pltpu.async_copy(msgs_hbm.at[pl.ds(sid * EPS_B + j * CB, CB)],
                         msg_v.at[q], semm[q])

    def _drain(q):
        pltpu.make_async_copy(msgs_hbm.at[pl.ds(0, CB)],
                              msg_v.at[q], semm[q]).wait()

    def _body(j, p, fire_next):
        if fire_next:
            _fire(j + 1, 1 - p)
        for t in range(CB // 16):
            d = dst_v[pl.ds(j * CB + t * 16, 16)]
            li = d - base
            ok = (li >= 0) & (li < HALF)
            idx_v[pl.ds(t * 16, 16)] = jnp.where(ok, li, HALF)
        _drain(p)
        pltpu.sync_copy(msg_v.at[p], agg_sh.at[idx_v], add=True)

    _fire(0, 0)

    def _pair(j2, _):
        j = j2 * 2
        _body(j, 0, True)
        _body(j + 1, 1, True)
        return 0
    lax.fori_loop(0, NCHUNK_B // 2, _pair, 0)
    _body(NCHUNK_B - 1, 0, False)

    plsc.subcore_barrier()
    pltpu.sync_copy(agg_sh.at[pl.ds(sid * BSTAGE, BSTAGE)], stage_v)
    pltpu.sync_copy(stage_v, out_hbm.at[cid].at[pl.ds(sid * BSTAGE, BSTAGE)])


@jax.jit
def _sc_agg(msgs, dst):
    mesh = plsc.VectorSubcoreMesh(core_axis_name="c", subcore_axis_name="s",
                                  num_cores=NC, num_subcores=NS)
    f = pl.kernel(
        _sc_agg_body,
        out_type=jax.ShapeDtypeStruct((NC, AGGR, AGG_W), jnp.float32),
        mesh=mesh,
        scratch_types=[
            pltpu.VMEM((2, CB, AGG_W), jnp.float32),
            pltpu.VMEM((EPS_B,), jnp.int32),
            pltpu.VMEM((CB,), jnp.int32),
            pltpu.VMEM((BSTAGE, AGG_W), jnp.float32),
            pltpu.VMEM_SHARED((AGGR, AGG_W), jnp.float32),
            pltpu.SemaphoreType.DMA,
            pltpu.SemaphoreType.DMA,
        ],
        compiler_params=pltpu.CompilerParams(use_tc_tiling_on_sc=False),
    )
    return f(msgs, dst)


def _final_kernel(agg_ref, xh_ref, root_ref, bias_ref, out_ref):
    agg = agg_ref[...]
    deg = jnp.maximum(agg[:, D_H:D_H + 1], 1.0)
    a = agg[:, :D_H] / deg
    r = jnp.dot(xh_ref[...], root_ref[...], preferred_element_type=jnp.float32)
    out_ref[...] = jnp.tanh(a + r + bias_ref[...])


@jax.jit
def _final(agg, xh, rootc, bias2):
    BF = 2000
    return pl.pallas_call(
        _final_kernel,
        grid=(N // BF,),
        in_specs=[
            pl.BlockSpec((BF, AGG_W), lambda i: (i, 0)),
            pl.BlockSpec((BF, D_IN + D_H), lambda i: (i, 0)),
            pl.BlockSpec((D_IN + D_H, D_H), lambda i: (0, 0)),
            pl.BlockSpec((1, D_H), lambda i: (0, 0)),
        ],
        out_specs=pl.BlockSpec((BF, D_H), lambda i: (i, 0)),
        out_shape=jax.ShapeDtypeStruct((N, D_H), jnp.float32),
    )(agg, xh, rootc, bias2)


def kernel(x, hidden, edge_index, edge_attr,
           W_xr, root_xr, bias_xr, W_hr, root_hr, bias_hr):
    xh = jnp.concatenate([x, hidden], axis=1)
    wx = W_xr.transpose(1, 0, 2)[:, :, _PERM].reshape(D_IN, K * D_H)
    wh = W_hr.transpose(1, 0, 2)[:, :, _PERM].reshape(D_H, K * D_H)
    w2 = jnp.concatenate([wx, wh], axis=0)
    w2 = jnp.pad(w2, ((0, 0), (0, (KPAD - K) * D_H)))
    table = _matmul(xh, w2).reshape(N * KPAD, D_H).astype(jnp.bfloat16)

    src_t = jnp.pad(edge_index[0:1].astype(jnp.int32), ((0, 7), (0, 0)))
    dst = edge_index[1].astype(jnp.int32)
    attr_t = jnp.pad(edge_attr.T, ((0, 8 - DIM), (0, 0)))
    basis_t, gidx_t = _edge_prep(attr_t, src_t)

    msgs = _sc_msg(gidx_t, basis_t.T, table)
    aggp = _sc_agg(msgs, dst)
    agg = jnp.concatenate([aggp[0, :HALF], aggp[1, :HALF]], axis=0)[:N]

    rootc = jnp.concatenate([root_xr, root_hr], axis=0)
    bias2 = (bias_xr + bias_hr).reshape(1, D_H)
    return _final(agg, xh, rootc, bias2)


# speedup vs baseline: 1.1302x; 1.1302x over previous
import jax
import jax.numpy as jnp
import numpy as np
from jax import lax
from jax.experimental import pallas as pl
from jax.experimental.pallas import tpu as pltpu
from jax.experimental.pallas import tpu_sc as plsc

N = 10000
E = 160000
D_IN = 128
D_H = 64
DIM = 4
KS = 3
K = KS ** DIM
KPAD = 84
S = 2 ** DIM
AGG_W = 80

NC, NS = 2, 16
NW = NC * NS
EPW = E // NW
C = 40
NCHUNK = EPW // C
ROWS = C * S
IDXW = 80

HALF = 5120
AGGR = 5248
BSTAGE = AGGR // NS
CB = 80
EPS_B = E // NS
NCHUNK_B = EPS_B // CB

_PERM = np.empty(D_H, dtype=np.int32)
for _p in range(D_H):
    _g, _q = divmod(_p, 32)
    _PERM[_p] = 32 * _g + (_q // 2 if _q % 2 == 0 else 16 + _q // 2)
_MASKHI = -65536


def _matmul_kernel(xh_ref, w_ref, out_ref):
    out_ref[...] = jnp.dot(xh_ref[...], w_ref[...],
                           preferred_element_type=jnp.float32
                           ).astype(jnp.bfloat16)


@jax.jit
def _matmul(xh, w2):
    BN, BM = 2000, 2688
    return pl.pallas_call(
        _matmul_kernel,
        grid=(N // BN, (KPAD * D_H) // BM),
        in_specs=[
            pl.BlockSpec((BN, D_IN + D_H), lambda i, j: (i, 0)),
            pl.BlockSpec((D_IN + D_H, BM), lambda i, j: (0, j)),
        ],
        out_specs=pl.BlockSpec((BN, BM), lambda i, j: (i, j)),
        out_shape=jax.ShapeDtypeStruct((N, KPAD * D_H), jnp.bfloat16),
    )(xh, w2)


def _edge_prep_kernel(attr_ref, src_ref, basis_ref, gidx_ref):
    attr = attr_ref[...][:DIM]
    src = src_ref[...][0:1]
    BEL = attr.shape[1]
    s_ids = lax.broadcasted_iota(jnp.int32, (S, BEL), 0)
    v = attr * (KS - 1)
    bot = jnp.clip(jnp.floor(v), 0.0, KS - 1.0)
    frac = v - bot
    boti = bot.astype(jnp.int32)
    topi = jnp.minimum(boti + 1, KS - 1)
    basis = jnp.ones((S, BEL), jnp.float32)
    wi = jnp.zeros((S, BEL), jnp.int32)
    for i in range(DIM):
        bit = (s_ids >> i) & 1
        f = frac[i:i + 1, :]
        basis = basis * jnp.where(bit == 1, f, 1.0 - f)
        idx = jnp.where(bit == 1, topi[i:i + 1, :], boti[i:i + 1, :])
        wi = wi + idx * (KS ** i)
    basis_ref[...] = basis
    gidx_ref[...] = src * KPAD + wi


@jax.jit
def _edge_prep(attr_t, src_t):
    BEL = 6400
    return pl.pallas_call(
        _edge_prep_kernel,
        grid=(E // BEL,),
        in_specs=[
            pl.BlockSpec((8, BEL), lambda i: (0, i)),
            pl.BlockSpec((8, BEL), lambda i: (0, i)),
        ],
        out_specs=[
            pl.BlockSpec((S, BEL), lambda i: (0, i)),
            pl.BlockSpec((S, BEL), lambda i: (0, i)),
        ],
        out_shape=[
            jax.ShapeDtypeStruct((S, E), jnp.float32),
            jax.ShapeDtypeStruct((S, E), jnp.int32),
        ],
    )(attr_t, src_t)


def _sc_msg_body(gidx_hbm, basis_hbm, table_hbm, msgs_hbm,
                 idx_v, basis_v, rows_v, msg_v,
                 semg0, semg1, seml0, seml1, semw0, semw1):
    cid = lax.axis_index("c")
    sid = lax.axis_index("s")
    wid = cid * NS + sid
    semg = (semg0, semg1)
    seml = (seml0, seml1)
    semw = (semw0, semw1)
    NIR = ROWS // IDXW

    deg_vec = jnp.where(lax.iota(jnp.int32, 16) == 0, 1.0, 0.0)

    def _init_msg(c, _):
        msg_v[0, c, pl.ds(D_H, 16)] = deg_vec
        msg_v[1, c, pl.ds(D_H, 16)] = deg_vec
        return 0
    lax.fori_loop(0, C, _init_msg, 0)

    def _drain_write(p):
        pltpu.make_async_copy(msg_v.at[p],
                              msgs_hbm.at[pl.ds(0, C)], semw[p]).wait()

    def _fire_loads(j, q):
        e0 = wid * EPW + j * C
        pltpu.async_copy(gidx_hbm.at[:, pl.ds(e0, C)], idx_v.at[q], seml[q])
        pltpu.async_copy(basis_hbm.at[pl.ds(e0, C)], basis_v.at[q], seml[q])

    def _drain_loads(q):
        pltpu.make_async_copy(gidx_hbm.at[:, pl.ds(0, C)],
                              idx_v.at[q], seml[q]).wait()
        pltpu.make_async_copy(basis_hbm.at[pl.ds(0, C)],
                              basis_v.at[q], seml[q]).wait()

    def _fire_gathers(q):
        for i in range(S):
            pltpu.async_copy(table_hbm.at[idx_v.at[q].at[i]],
                             rows_v.at[q].at[pl.ds(i * C, C)], semg[q])

    def _drain_gathers(q):
        for i in range(S):
            pltpu.make_async_copy(table_hbm.at[idx_v.at[q].at[i]],
                                  rows_v.at[q].at[pl.ds(i * C, C)],
                                  semg[q]).wait()

    def _compute(j, p):
        def _edge(c, _):
            bvec = basis_v[p, c, pl.ds(0, S)]
            a0 = jnp.zeros((16,), jnp.float32)
            a1 = a0
            a2 = a0
            a3 = a0
            for s in range(S):
                b = bvec[s]
                r = s * C + c
                v0 = plsc.bitcast(rows_v[p, r, pl.ds(0, 32)], jnp.int32)
                v1 = plsc.bitcast(rows_v[p, r, pl.ds(32, 32)], jnp.int32)
                a0 = a0 + b * plsc.bitcast(v0 << 16, jnp.float32)
                a1 = a1 + b * plsc.bitcast(v0 & _MASKHI, jnp.float32)
                a2 = a2 + b * plsc.bitcast(v1 << 16, jnp.float32)
                a3 = a3 + b * plsc.bitcast(v1 & _MASKHI, jnp.float32)
            msg_v[p, c, pl.ds(0, 16)] = a0
            msg_v[p, c, pl.ds(16, 16)] = a1
            msg_v[p, c, pl.ds(32, 16)] = a2
            msg_v[p, c, pl.ds(48, 16)] = a3
            return 0
        lax.fori_loop(0, C, _edge, 0)
        pltpu.async_copy(msg_v.at[p],
                         msgs_hbm.at[pl.ds(wid * EPW + j * C, C)], semw[p])

    def _body(j, p, fire_next, drain_write):
        q = 1 - p
        if fire_next:
            _fire_loads(j + 1, q)
        _drain_gathers(p)
        if drain_write:
            _drain_write(p)
        _compute(j, p)
        if fire_next:
            _drain_loads(q)
            _fire_gathers(q)

    _fire_loads(0, 0)
    _drain_loads(0)
    _fire_gathers(0)
    _body(0, 0, True, False)
    _body(1, 1, True, False)

    def _pair(j2, _):
        j = 2 + j2 * 2
        _body(j, 0, True, True)
        _body(j + 1, 1, True, True)
        return 0
    lax.fori_loop(0, (NCHUNK - 3) // 2, _pair, 0)
    _body(NCHUNK - 1, 0, False, True)
    _drain_write(0)
    _drain_write(1)


@jax.jit
def _sc_msg(gidx2, basis, table):
    mesh = plsc.VectorSubcoreMesh(core_axis_name="c", subcore_axis_name="s",
                                  num_cores=NC, num_subcores=NS)
    f = pl.kernel(
        _sc_msg_body,
        out_type=jax.ShapeDtypeStruct((E, AGG_W), jnp.float32),
        mesh=mesh,
        scratch_types=[
            pltpu.VMEM((2, S, C), jnp.int32),
            pltpu.VMEM((2, C, S), jnp.float32),
            pltpu.VMEM((2, ROWS, D_H), jnp.bfloat16),
            pltpu.VMEM((2, C, AGG_W), jnp.float32),
            pltpu.SemaphoreType.DMA,
            pltpu.SemaphoreType.DMA,
            pltpu.SemaphoreType.DMA,
            pltpu.SemaphoreType.DMA,
            pltpu.SemaphoreType.DMA,
            pltpu.SemaphoreType.DMA,
        ],
        compiler_params=pltpu.CompilerParams(use_tc_tiling_on_sc=False,
                                             needs_layout_passes=False),
    )
    return f(gidx2, basis, table)


def _sc_agg_body(msgs_hbm, dst_hbm, out_hbm,
                 msg_v, dst_v, idx_v, stage_v, agg_sh,
                 semm0, semm1, sems0, sems1):
    cid = lax.axis_index("c")
    sid = lax.axis_index("s")
    base = cid * HALF
    semm = (semm0, semm1)
    sems = (sems0, sems1)

    zero16 = jnp.zeros((16,), jnp.float32)

    def _zero_row(i, _):
        for j in range(AGG_W // 16):
            stage_v[i, pl.ds(j * 16, 16)] = zero16
        return 0
    lax.fori_loop(0, BSTAGE, _zero_row, 0)
    pltpu.sync_copy(stage_v, agg_sh.at[pl.ds(sid * BSTAGE, BSTAGE)])
    pltpu.sync_copy(dst_hbm.at[pl.ds(sid * EPS_B, EPS_B)], dst_v)
    plsc.subcore_barrier()

    def _fire(j, q):
        pltpu.async_copy(msgs_hbm.at[pl.ds(sid * EPS_B + j * CB, CB)],
                         msg_v.at[q], semm[q])

    def _drain(q):
        pltpu.make_async_copy(msgs_hbm.at[pl.ds(0, CB)],
                              msg_v.at[q], semm[q]).wait()

    def _drain_scatter(q):
        pltpu.make_async_copy(msg_v.at[q],
                              agg_sh.at[idx_v.at[q]], sems[q]).wait()

    def _body(j, p, fire_next, drain_scat):
        if fire_next:
            _fire(j + 1, 1 - p)
        if drain_scat:
            _drain_scatter(p)
        for t in range(CB // 16):
            d = dst_v[pl.ds(j * CB + t * 16, 16)]
            li = d - base
            ok = (li >= 0) & (li < HALF)
            idx_v[p, pl.ds(t * 16, 16)] = jnp.where(ok, li, HALF)
        _drain(p)
        pltpu.async_copy(msg_v.at[p], agg_sh.at[idx_v.at[p]], sems[p],
                         add=True)

    _fire(0, 0)
    _body(0, 0, True, False)
    _body(1, 1, True, False)

    def _pair(j2, _):
        j = 2 + j2 * 2
        _body(j, 0, True, True)
        _body(j + 1, 1, True, True)
        return 0
    lax.fori_loop(0, (NCHUNK_B - 3) // 2, _pair, 0)
    _body(NCHUNK_B - 1, 0, False, True)
    _drain_scatter(1)
    _drain_scatter(0)

    plsc.subcore_barrier()
    pltpu.sync_copy(agg_sh.at[pl.ds(sid * BSTAGE, BSTAGE)], stage_v)
    pltpu.sync_copy(stage_v, out_hbm.at[cid].at[pl.ds(sid * BSTAGE, BSTAGE)])


@jax.jit
def _sc_agg(msgs, dst):
    mesh = plsc.VectorSubcoreMesh(core_axis_name="c", subcore_axis_name="s",
                                  num_cores=NC, num_subcores=NS)
    f = pl.kernel(
        _sc_agg_body,
        out_type=jax.ShapeDtypeStruct((NC, AGGR, AGG_W), jnp.float32),
        mesh=mesh,
        scratch_types=[
            pltpu.VMEM((2, CB, AGG_W), jnp.float32),
            pltpu.VMEM((EPS_B,), jnp.int32),
            pltpu.VMEM((2, CB), jnp.int32),
            pltpu.VMEM((BSTAGE, AGG_W), jnp.float32),
            pltpu.VMEM_SHARED((AGGR, AGG_W), jnp.float32),
            pltpu.SemaphoreType.DMA,
            pltpu.SemaphoreType.DMA,
            pltpu.SemaphoreType.DMA,
            pltpu.SemaphoreType.DMA,
        ],
        compiler_params=pltpu.CompilerParams(use_tc_tiling_on_sc=False),
    )
    return f(msgs, dst)


def _final_kernel(agg_ref, xh_ref, root_ref, bias_ref, out_ref):
    agg = agg_ref[...]
    deg = jnp.maximum(agg[:, D_H:D_H + 1], 1.0)
    a = agg[:, :D_H] / deg
    r = jnp.dot(xh_ref[...], root_ref[...], preferred_element_type=jnp.float32)
    out_ref[...] = jnp.tanh(a + r + bias_ref[...])


@jax.jit
def _final(agg, xh, rootc, bias2):
    BF = 2000
    return pl.pallas_call(
        _final_kernel,
        grid=(N // BF,),
        in_specs=[
            pl.BlockSpec((BF, AGG_W), lambda i: (i, 0)),
            pl.BlockSpec((BF, D_IN + D_H), lambda i: (i, 0)),
            pl.BlockSpec((D_IN + D_H, D_H), lambda i: (0, 0)),
            pl.BlockSpec((1, D_H), lambda i: (0, 0)),
        ],
        out_specs=pl.BlockSpec((BF, D_H), lambda i: (i, 0)),
        out_shape=jax.ShapeDtypeStruct((N, D_H), jnp.float32),
    )(agg, xh, rootc, bias2)


def kernel(x, hidden, edge_index, edge_attr,
           W_xr, root_xr, bias_xr, W_hr, root_hr, bias_hr):
    xh = jnp.concatenate([x, hidden], axis=1)
    wx = W_xr.transpose(1, 0, 2)[:, :, _PERM].reshape(D_IN, K * D_H)
    wh = W_hr.transpose(1, 0, 2)[:, :, _PERM].reshape(D_H, K * D_H)
    w2 = jnp.concatenate([wx, wh], axis=0)
    w2 = jnp.pad(w2, ((0, 0), (0, (KPAD - K) * D_H)))
    table = _matmul(xh, w2).reshape(N * KPAD, D_H)

    src_t = jnp.pad(edge_index[0:1].astype(jnp.int32), ((0, 7), (0, 0)))
    dst = edge_index[1].astype(jnp.int32)
    attr_t = jnp.pad(edge_attr.T, ((0, 8 - DIM), (0, 0)))
    basis_t, gidx_t = _edge_prep(attr_t, src_t)

    msgs = _sc_msg(gidx_t, basis_t.T, table)
    aggp = _sc_agg(msgs, dst)
    agg = jnp.concatenate([aggp[0, :HALF], aggp[1, :HALF]], axis=0)[:N]

    rootc = jnp.concatenate([root_xr, root_hr], axis=0)
    bias2 = (bias_xr + bias_hr).reshape(1, D_H)
    return _final(agg, xh, rootc, bias2)


# speedup vs baseline: 1.1319x; 1.0015x over previous
import jax
import jax.numpy as jnp
import numpy as np
from jax import lax
from jax.experimental import pallas as pl
from jax.experimental.pallas import tpu as pltpu
from jax.experimental.pallas import tpu_sc as plsc

N = 10000
E = 160000
D_IN = 128
D_H = 64
DIM = 4
KS = 3
K = KS ** DIM
KPAD = 84
S = 2 ** DIM
AGG_W = 80

NC, NS = 2, 16
NW = NC * NS
EPW = E // NW
C = 40
NCHUNK = EPW // C
ROWS = C * S
IDXW = 80

HALF = 5120
AGGR = 5248
BSTAGE = AGGR // NS
CB = 80
EPS_B = E // NS
NCHUNK_B = EPS_B // CB

_PERM = np.empty(D_H, dtype=np.int32)
for _p in range(D_H):
    _g, _q = divmod(_p, 32)
    _PERM[_p] = 32 * _g + (_q // 2 if _q % 2 == 0 else 16 + _q // 2)
_MASKHI = -65536


def _matmul_kernel(xh_ref, w_ref, out_ref):
    out_ref[...] = jnp.dot(xh_ref[...], w_ref[...],
                           preferred_element_type=jnp.float32
                           ).astype(jnp.bfloat16)


@jax.jit
def _matmul(xh, w2):
    BN, BM = 2000, 2688
    return pl.pallas_call(
        _matmul_kernel,
        grid=(N // BN, (KPAD * D_H) // BM),
        in_specs=[
            pl.BlockSpec((BN, D_IN + D_H), lambda i, j: (i, 0)),
            pl.BlockSpec((D_IN + D_H, BM), lambda i, j: (0, j)),
        ],
        out_specs=pl.BlockSpec((BN, BM), lambda i, j: (i, j)),
        out_shape=jax.ShapeDtypeStruct((N, KPAD * D_H), jnp.bfloat16),
    )(xh, w2)


def _edge_prep_kernel(attr_ref, src_ref, basis_ref, gidx_ref):
    attr = attr_ref[...][:DIM]
    src = src_ref[...][0:1]
    BEL = attr.shape[1]
    s_ids = lax.broadcasted_iota(jnp.int32, (S, BEL), 0)
    v = attr * (KS - 1)
    bot = jnp.clip(jnp.floor(v), 0.0, KS - 1.0)
    frac = v - bot
    boti = bot.astype(jnp.int32)
    topi = jnp.minimum(boti + 1, KS - 1)
    basis = jnp.ones((S, BEL), jnp.float32)
    wi = jnp.zeros((S, BEL), jnp.int32)
    for i in range(DIM):
        bit = (s_ids >> i) & 1
        f = frac[i:i + 1, :]
        basis = basis * jnp.where(bit == 1, f, 1.0 - f)
        idx = jnp.where(bit == 1, topi[i:i + 1, :], boti[i:i + 1, :])
        wi = wi + idx * (KS ** i)
    basis_ref[...] = basis
    gidx_ref[...] = src * KPAD + wi


@jax.jit
def _edge_prep(attr_t, src_t):
    BEL = 6400
    return pl.pallas_call(
        _edge_prep_kernel,
        grid=(E // BEL,),
        in_specs=[
            pl.BlockSpec((8, BEL), lambda i: (0, i)),
            pl.BlockSpec((8, BEL), lambda i: (0, i)),
        ],
        out_specs=[
            pl.BlockSpec((S, BEL), lambda i: (0, i)),
            pl.BlockSpec((S, BEL), lambda i: (0, i)),
        ],
        out_shape=[
            jax.ShapeDtypeStruct((S, E), jnp.float32),
            jax.ShapeDtypeStruct((S, E), jnp.int32),
        ],
    )(attr_t, src_t)


def _sc_msg_body(gidx_hbm, basis_hbm, table_hbm, msgs_hbm,
                 idx_v, basis_v, rows_v, msg_v,
                 semg0, semg1, seml0, seml1, semw0, semw1):
    cid = lax.axis_index("c")
    sid = lax.axis_index("s")
    wid = cid * NS + sid
    semg = (semg0, semg1)
    seml = (seml0, seml1)
    semw = (semw0, semw1)
    NIR = ROWS // IDXW

    deg_vec = jnp.where(lax.iota(jnp.int32, 16) == 0, 1.0, 0.0)

    def _init_msg(c, _):
        msg_v[0, c, pl.ds(D_H, 16)] = deg_vec
        msg_v[1, c, pl.ds(D_H, 16)] = deg_vec
        return 0
    lax.fori_loop(0, C, _init_msg, 0)

    def _drain_write(p):
        pltpu.make_async_copy(msg_v.at[p],
                              msgs_hbm.at[pl.ds(0, C)], semw[p]).wait()

    def _fire_loads(j, q):
        e0 = wid * EPW + j * C
        pltpu.async_copy(gidx_hbm.at[:, pl.ds(e0, C)], idx_v.at[q], seml[q])
        pltpu.async_copy(basis_hbm.at[pl.ds(e0, C)], basis_v.at[q], seml[q])

    def _drain_loads(q):
        pltpu.make_async_copy(gidx_hbm.at[:, pl.ds(0, C)],
                              idx_v.at[q], seml[q]).wait()
        pltpu.make_async_copy(basis_hbm.at[pl.ds(0, C)],
                              basis_v.at[q], seml[q]).wait()

    def _fire_gathers(q):
        for i in range(S):
            pltpu.async_copy(table_hbm.at[idx_v.at[q].at[i]],
                             rows_v.at[q].at[pl.ds(i * C, C)], semg[q])

    def _drain_gathers(q):
        for i in range(S):
            pltpu.make_async_copy(table_hbm.at[idx_v.at[q].at[i]],
                                  rows_v.at[q].at[pl.ds(i * C, C)],
                                  semg[q]).wait()

    def _compute(j, p):
        def _edge(c, _):
            bvec = basis_v[p, c, pl.ds(0, S)]
            a0 = jnp.zeros((16,), jnp.float32)
            a1 = a0
            a2 = a0
            a3 = a0
            for s in range(S):
                b = bvec[s]
                r = s * C + c
                v0 = plsc.bitcast(rows_v[p, r, pl.ds(0, 32)], jnp.int32)
                v1 = plsc.bitcast(rows_v[p, r, pl.ds(32, 32)], jnp.int32)
                a0 = a0 + b * plsc.bitcast(v0 << 16, jnp.float32)
                a1 = a1 + b * plsc.bitcast(v0 & _MASKHI, jnp.float32)
                a2 = a2 + b * plsc.bitcast(v1 << 16, jnp.float32)
                a3 = a3 + b * plsc.bitcast(v1 & _MASKHI, jnp.float32)
            msg_v[p, c, pl.ds(0, 16)] = a0
            msg_v[p, c, pl.ds(16, 16)] = a1
            msg_v[p, c, pl.ds(32, 16)] = a2
            msg_v[p, c, pl.ds(48, 16)] = a3
            return 0
        lax.fori_loop(0, C, _edge, 0)
        pltpu.async_copy(msg_v.at[p],
                         msgs_hbm.at[pl.ds(wid * EPW + j * C, C)], semw[p])

    def _body(j, p, fire_next, drain_write):
        q = 1 - p
        if fire_next:
            _fire_loads(j + 1, q)
        _drain_gathers(p)
        if drain_write:
            _drain_write(p)
        _compute(j, p)
        if fire_next:
            _drain_loads(q)
            _fire_gathers(q)

    _fire_loads(0, 0)
    _drain_loads(0)
    _fire_gathers(0)
    _body(0, 0, True, False)
    _body(1, 1, True, False)

    def _pair(j2, _):
        j = 2 + j2 * 2
        _body(j, 0, True, True)
        _body(j + 1, 1, True, True)
        return 0
    lax.fori_loop(0, (NCHUNK - 3) // 2, _pair, 0)
    _body(NCHUNK - 1, 0, False, True)
    _drain_write(0)
    _drain_write(1)


@jax.jit
def _sc_msg(gidx2, basis, table):
    mesh = plsc.VectorSubcoreMesh(core_axis_name="c", subcore_axis_name="s",
                                  num_cores=NC, num_subcores=NS)
    f = pl.kernel(
        _sc_msg_body,
        out_type=jax.ShapeDtypeStruct((E, AGG_W), jnp.float32),
        mesh=mesh,
        scratch_types=[
            pltpu.VMEM((2, S, C), jnp.int32),
            pltpu.VMEM((2, C, S), jnp.float32),
            pltpu.VMEM((2, ROWS, D_H), jnp.bfloat16),
            pltpu.VMEM((2, C, AGG_W), jnp.float32),
            pltpu.SemaphoreType.DMA,
            pltpu.SemaphoreType.DMA,
            pltpu.SemaphoreType.DMA,
            pltpu.SemaphoreType.DMA,
            pltpu.SemaphoreType.DMA,
            pltpu.SemaphoreType.DMA,
        ],
        compiler_params=pltpu.CompilerParams(use_tc_tiling_on_sc=False,
                                             needs_layout_passes=False),
    )
    return f(gidx2, basis, table)


def _sc_agg_body(msgs_hbm, dst_hbm, out_hbm,
                 msg_v, dst_v, idx_v, stage_v, agg_sh,
                 semm0, semm1, sems0, sems1):
    cid = lax.axis_index("c")
    sid = lax.axis_index("s")
    base = cid * HALF
    semm = (semm0, semm1)
    sems = (sems0, sems1)

    zero16 = jnp.zeros((16,), jnp.float32)

    def _zero_row(i, _):
        for j in range(AGG_W // 16):
            stage_v[i, pl.ds(j * 16, 16)] = zero16
        return 0
    lax.fori_loop(0, BSTAGE, _zero_row, 0)
    pltpu.sync_copy(stage_v, agg_sh.at[pl.ds(sid * BSTAGE, BSTAGE)])
    pltpu.sync_copy(dst_hbm.at[pl.ds(sid * EPS_B, EPS_B)], dst_v)
    plsc.subcore_barrier()

    def _fire(j, q):
        pltpu.async_copy(msgs_hbm.at[pl.ds(sid * EPS_B + j * CB, CB)],
                         msg_v.at[q], semm[q])

    def _drain(q):
        pltpu.make_async_copy(msgs_hbm.at[pl.ds(0, CB)],
                              msg_v.at[q], semm[q]).wait()

    def _drain_scatter(q):
        pltpu.make_async_copy(msg_v.at[q],
                              agg_sh.at[idx_v.at[q]], sems[q]).wait()

    def _body(j, p, fire_next, drain_scat):
        q = 1 - p
        if drain_scat:
            _drain_scatter(q)
        if fire_next:
            _fire(j + 1, q)
        for t in range(CB // 16):
            d = dst_v[pl.ds(j * CB + t * 16, 16)]
            li = d - base
            ok = (li >= 0) & (li < HALF)
            idx_v[p, pl.ds(t * 16, 16)] = jnp.where(ok, li, HALF)
        _drain(p)
        pltpu.async_copy(msg_v.at[p], agg_sh.at[idx_v.at[p]], sems[p],
                         add=True)

    _fire(0, 0)
    _body(0, 0, True, False)

    def _pair(j2, _):
        j = 1 + j2 * 2
        _body(j, 1, True, True)
        _body(j + 1, 0, True, True)
        return 0
    lax.fori_loop(0, (NCHUNK_B - 3) // 2, _pair, 0)
    _body(NCHUNK_B - 2, 1, True, True)
    _body(NCHUNK_B - 1, 0, False, True)
    _drain_scatter(0)

    plsc.subcore_barrier()
    pltpu.sync_copy(agg_sh.at[pl.ds(sid * BSTAGE, BSTAGE)], stage_v)
    pltpu.sync_copy(stage_v, out_hbm.at[cid].at[pl.ds(sid * BSTAGE, BSTAGE)])


@jax.jit
def _sc_agg(msgs, dst):
    mesh = plsc.VectorSubcoreMesh(core_axis_name="c", subcore_axis_name="s",
                                  num_cores=NC, num_subcores=NS)
    f = pl.kernel(
        _sc_agg_body,
        out_type=jax.ShapeDtypeStruct((NC, AGGR, AGG_W), jnp.float32),
        mesh=mesh,
        scratch_types=[
            pltpu.VMEM((2, CB, AGG_W), jnp.float32),
            pltpu.VMEM((EPS_B,), jnp.int32),
            pltpu.VMEM((2, CB), jnp.int32),
            pltpu.VMEM((BSTAGE, AGG_W), jnp.float32),
            pltpu.VMEM_SHARED((AGGR, AGG_W), jnp.float32),
            pltpu.SemaphoreType.DMA,
            pltpu.SemaphoreType.DMA,
            pltpu.SemaphoreType.DMA,
            pltpu.SemaphoreType.DMA,
        ],
        compiler_params=pltpu.CompilerParams(use_tc_tiling_on_sc=False),
    )
    return f(msgs, dst)


def _final_kernel(agg_ref, xh_ref, root_ref, bias_ref, out_ref):
    agg = agg_ref[...]
    deg = jnp.maximum(agg[:, D_H:D_H + 1], 1.0)
    a = agg[:, :D_H] / deg
    r = jnp.dot(xh_ref[...], root_ref[...], preferred_element_type=jnp.float32)
    out_ref[...] = jnp.tanh(a + r + bias_ref[...])


@jax.jit
def _final(agg, xh, rootc, bias2):
    BF = 2000
    return pl.pallas_call(
        _final_kernel,
        grid=(N // BF,),
        in_specs=[
            pl.BlockSpec((BF, AGG_W), lambda i: (i, 0)),
            pl.BlockSpec((BF, D_IN + D_H), lambda i: (i, 0)),
            pl.BlockSpec((D_IN + D_H, D_H), lambda i: (0, 0)),
            pl.BlockSpec((1, D_H), lambda i: (0, 0)),
        ],
        out_specs=pl.BlockSpec((BF, D_H), lambda i: (i, 0)),
        out_shape=jax.ShapeDtypeStruct((N, D_H), jnp.float32),
    )(agg, xh, rootc, bias2)


def kernel(x, hidden, edge_index, edge_attr,
           W_xr, root_xr, bias_xr, W_hr, root_hr, bias_hr):
    xh = jnp.concatenate([x, hidden], axis=1)
    wx = W_xr.transpose(1, 0, 2)[:, :, _PERM].reshape(D_IN, K * D_H)
    wh = W_hr.transpose(1, 0, 2)[:, :, _PERM].reshape(D_H, K * D_H)
    w2 = jnp.concatenate([wx, wh], axis=0)
    w2 = jnp.pad(w2, ((0, 0), (0, (KPAD - K) * D_H)))
    table = _matmul(xh, w2).reshape(N * KPAD, D_H)

    src_t = jnp.pad(edge_index[0:1].astype(jnp.int32), ((0, 7), (0, 0)))
    dst = edge_index[1].astype(jnp.int32)
    attr_t = jnp.pad(edge_attr.T, ((0, 8 - DIM), (0, 0)))
    basis_t, gidx_t = _edge_prep(attr_t, src_t)

    msgs = _sc_msg(gidx_t, basis_t.T, table)
    aggp = _sc_agg(msgs, dst)
    agg = jnp.concatenate([aggp[0, :HALF], aggp[1, :HALF]], axis=0)[:N]

    rootc = jnp.concatenate([root_xr, root_hr], axis=0)
    bias2 = (bias_xr + bias_hr).reshape(1, D_H)
    return _final(agg, xh, rootc, bias2)
